# baseline (speedup 1.0000x reference)
import functools

import jax
import jax.numpy as jnp
from jax import lax
from jax.experimental import pallas as pl
from jax.experimental.pallas import tpu as pltpu
from jax.experimental.pallas import tpu_sc as plsc

_NUM_VOCAB = 1000000
_EMBED_DIM = 64
_BATCH = 4096
_HIST = 200

_INFO = plsc.get_sparse_core_info()
_NC, _NS = _INFO.num_cores, _INFO.num_subcores
_NW = _NC * _NS
_BPW = _BATCH // _NW
_CH = (128, 72)
_H0 = (0, 128)


_NSTEPS = 2 * _BPW

_TBLK = 128
_NFULL = _NUM_VOCAB // _TBLK
_TAIL = _NUM_VOCAB - _NFULL * _TBLK


def _iota16():
    return lax.iota(jnp.int32, 16)


def _trans_body(tT_hbm, tail_hbm, out_hbm, inb0, inb1, outb0, outb1, tinb,
                tin0, tin1, tout0, tout1):
    wid = lax.axis_index("s") * _NC + lax.axis_index("c")
    inb = (inb0, inb1)
    outb = (outb0, outb1)
    tin = (tin0, tin1)
    tout = (tout0, tout1)
    nblk = 244 + jnp.where(wid < jnp.int32(_NFULL - 244 * _NW), 1, 0)

    def col0_of(i):
        return _TBLK * (wid + _NW * i)

    def load(i, p):
        pltpu.async_copy(tT_hbm.at[:, pl.ds(col0_of(i), _TBLK)], inb[p],
                         tin[p])

    def transpose_block(src, dst, width):
        def col8(j8, carry):
            for u in range(8):
                j = j8 * 8 + u
                for k in range(_EMBED_DIM // 16):
                    vec = plsc.load_gather(src, [_iota16() + 16 * k,
                                                 jnp.full((16,), j, jnp.int32)])
                    dst[j, pl.ds(16 * k, 16)] = vec
            return carry
        lax.fori_loop(0, width // 8, col8, 0)

    load(0, 0)

    @pl.when(nblk > 1)
    def _():
        load(1, 1)

    def blk_step(i, carry):
        for p in range(2):
            ii = i * 2 + p

            @pl.when(ii < nblk)
            def _():
                pltpu.make_async_copy(tT_hbm.at[:, pl.ds(0, _TBLK)], inb[p],
                                      tin[p]).wait()
                @pl.when(ii >= 2)
                def _():
                    pltpu.make_async_copy(outb[p],
                                          out_hbm.at[pl.ds(0, _TBLK)],
                                          tout[p]).wait()
                transpose_block(inb[p], outb[p], _TBLK)

                @pl.when(ii + 2 < nblk)
                def _():
                    load(ii + 2, p)
                pltpu.async_copy(outb[p], out_hbm.at[pl.ds(col0_of(ii), _TBLK)],
                                 tout[p])
        return carry

    lax.fori_loop(0, 123, blk_step, 0)

    @pl.when(nblk > 0)
    def _():
        pltpu.make_async_copy(outb[0], out_hbm.at[pl.ds(0, _TBLK)],
                              tout[0]).wait()

    @pl.when(nblk > 1)
    def _():
        pltpu.make_async_copy(outb[1], out_hbm.at[pl.ds(0, _TBLK)],
                              tout[1]).wait()

    @pl.when(wid == 0)
    def _():
        pltpu.sync_copy(tail_hbm, tinb)
        transpose_block(tinb, outb0, _TAIL)
        pltpu.sync_copy(outb0.at[pl.ds(0, _TAIL)],
                        out_hbm.at[pl.ds(_NFULL * _TBLK, _TAIL)])


def _body(x_hbm, table_hbm, out_hbm, idx_v,
          rows00, rows01, rows10, rows11,
          gs00, gs01, gs10, gs11, os00, os01, os10, os11):
    wid = lax.axis_index("s") * _NC + lax.axis_index("c")
    b0 = wid * _BPW
    pltpu.sync_copy(x_hbm.at[pl.ds(b0, _BPW)], idx_v)

    rows = ((rows00, rows01), (rows10, rows11))
    gs = ((gs00, gs01), (gs10, gs11))
    os = ((os00, os01), (os10, os11))

    def gather(s, p, b):
        pltpu.async_copy(table_hbm.at[idx_v.at[s // 2, pl.ds(_H0[b], _CH[b])]],
                         rows[p][b], gs[p][b])

    def wb_dst(r, b):
        return out_hbm.at[b0 + r, pl.ds(_H0[b], _CH[b]), pl.ds(0, _EMBED_DIM)]

    def quad_step(s4, carry):
        for k in range(4):
            b = k & 1
            p = (k >> 1) & 1
            s = s4 * 4 + k
            pltpu.make_async_copy(table_hbm.at[idx_v.at[0, pl.ds(0, _CH[b])]],
                                  rows[p][b], gs[p][b]).wait()
            pltpu.async_copy(rows[p][b], wb_dst(s // 2, b), os[p][b])

            @pl.when(s + 4 < _NSTEPS)
            def _():
                pltpu.make_async_copy(rows[p][b], wb_dst(0, b),
                                      os[p][b]).wait()
                gather(s + 4, p, b)
        return carry

    for k in range(4):
        gather(k, (k >> 1) & 1, k & 1)
    lax.fori_loop(0, _NSTEPS // 4, quad_step, 0)

    for p in range(2):
        for b in range(2):
            pltpu.make_async_copy(rows[p][b], wb_dst(0, b), os[p][b]).wait()


@jax.jit
def kernel(x, table):
    x2 = x * 2
    trans = functools.partial(
        pl.kernel,
        out_type=jax.ShapeDtypeStruct((_NUM_VOCAB, 2 * _EMBED_DIM),
                                      jnp.float32),
        mesh=plsc.VectorSubcoreMesh(core_axis_name="c", subcore_axis_name="s"),
        scratch_types=[
            pltpu.VMEM((_EMBED_DIM, _TBLK), jnp.float32),
            pltpu.VMEM((_EMBED_DIM, _TBLK), jnp.float32),
            pltpu.VMEM((_TBLK, 2 * _EMBED_DIM), jnp.float32),
            pltpu.VMEM((_TBLK, 2 * _EMBED_DIM), jnp.float32),
            pltpu.VMEM((_EMBED_DIM, _TAIL), jnp.float32),
        ] + [pltpu.SemaphoreType.DMA] * 4,
        compiler_params=pltpu.CompilerParams(use_tc_tiling_on_sc=True,
                                             needs_layout_passes=False),
    )(_trans_body)
    t128 = trans(table.T, table.T[:, _NFULL * _TBLK:])
    t2 = t128.reshape(2 * _NUM_VOCAB, _EMBED_DIM)
    run = functools.partial(
        pl.kernel,
        out_type=jax.ShapeDtypeStruct((_BATCH, _HIST, 2 * _EMBED_DIM),
                                      jnp.float32),
        mesh=plsc.VectorSubcoreMesh(core_axis_name="c", subcore_axis_name="s"),
        scratch_types=[
            pltpu.VMEM((_BPW, _HIST), jnp.int32),
            pltpu.VMEM((_CH[0], _EMBED_DIM), jnp.float32),
            pltpu.VMEM((_CH[1], _EMBED_DIM), jnp.float32),
            pltpu.VMEM((_CH[0], _EMBED_DIM), jnp.float32),
            pltpu.VMEM((_CH[1], _EMBED_DIM), jnp.float32),
        ] + [pltpu.SemaphoreType.DMA] * 8,
        compiler_params=pltpu.CompilerParams(use_tc_tiling_on_sc=False),
    )(_body)
    out_pad = run(x2, t2)
    return out_pad[:, :, :_EMBED_DIM]


# speedup vs baseline: 2.1778x; 2.1778x over previous
import functools

import jax
import jax.numpy as jnp
from jax import lax
from jax.experimental import pallas as pl
from jax.experimental.pallas import tpu as pltpu
from jax.experimental.pallas import tpu_sc as plsc

_NUM_VOCAB = 1000000
_EMBED_DIM = 64
_BATCH = 4096
_HIST = 200

_INFO = plsc.get_sparse_core_info()
_NC, _NS = _INFO.num_cores, _INFO.num_subcores
_NW = _NC * _NS
_BPW = _BATCH // _NW
_CH = (128, 72)
_H0 = (0, 128)


_NSTEPS = 2 * _BPW

_TBLK = 128
_NFULL = _NUM_VOCAB // _TBLK
_TAIL = _NUM_VOCAB - _NFULL * _TBLK


def _iota16():
    return lax.iota(jnp.int32, 16)


def _trans_body(tT_hbm, tail_hbm, out_hbm, inb0, inb1, outb0, outb1, tinb,
                tin0, tin1, tout0, tout1):
    wid = lax.axis_index("s") * _NC + lax.axis_index("c")
    inb = (inb0, inb1)
    outb = (outb0, outb1)
    tin = (tin0, tin1)
    tout = (tout0, tout1)
    nblk = 244 + jnp.where(wid < jnp.int32(_NFULL - 244 * _NW), 1, 0)

    def col0_of(i):
        return _TBLK * (wid + _NW * i)

    def load(i, p):
        pltpu.async_copy(tT_hbm.at[:, pl.ds(col0_of(i), _TBLK)], inb[p],
                         tin[p])

    def transpose_block(src, dst, width):
        iot = _iota16()

        def col8(j8, carry):
            for u in range(4):
                j = j8 * 4 + u
                cols = lax.bitwise_and(j + iot, jnp.int32(width - 1))
                for k in range(_EMBED_DIM // 16):
                    rows = iot + 16 * k
                    vec = plsc.load_gather(src, [rows, cols])
                    plsc.store_scatter(dst, [cols, rows], vec)
            return carry
        lax.fori_loop(0, width // 4, col8, 0)

    load(0, 0)

    @pl.when(nblk > 1)
    def _():
        load(1, 1)

    def blk_step(i, carry):
        for p in range(2):
            ii = i * 2 + p

            @pl.when(ii < nblk)
            def _():
                pltpu.make_async_copy(tT_hbm.at[:, pl.ds(0, _TBLK)], inb[p],
                                      tin[p]).wait()
                @pl.when(ii >= 2)
                def _():
                    pltpu.make_async_copy(outb[p],
                                          out_hbm.at[pl.ds(0, _TBLK)],
                                          tout[p]).wait()
                transpose_block(inb[p], outb[p], _TBLK)

                @pl.when(ii + 2 < nblk)
                def _():
                    load(ii + 2, p)
                pltpu.async_copy(outb[p], out_hbm.at[pl.ds(col0_of(ii), _TBLK)],
                                 tout[p])
        return carry

    lax.fori_loop(0, 123, blk_step, 0)

    @pl.when(nblk > 0)
    def _():
        pltpu.make_async_copy(outb[0], out_hbm.at[pl.ds(0, _TBLK)],
                              tout[0]).wait()

    @pl.when(nblk > 1)
    def _():
        pltpu.make_async_copy(outb[1], out_hbm.at[pl.ds(0, _TBLK)],
                              tout[1]).wait()

    @pl.when(wid == 0)
    def _():
        pltpu.sync_copy(tail_hbm, tinb)
        transpose_block(tinb, outb0, _TAIL)
        pltpu.sync_copy(outb0.at[pl.ds(0, _TAIL)],
                        out_hbm.at[pl.ds(_NFULL * _TBLK, _TAIL)])


def _body(x_hbm, table_hbm, out_hbm, idx_v,
          rows00, rows01, rows10, rows11,
          gs00, gs01, gs10, gs11, os00, os01, os10, os11):
    wid = lax.axis_index("s") * _NC + lax.axis_index("c")
    b0 = wid * _BPW
    pltpu.sync_copy(x_hbm.at[pl.ds(b0, _BPW)], idx_v)

    rows = ((rows00, rows01), (rows10, rows11))
    gs = ((gs00, gs01), (gs10, gs11))
    os = ((os00, os01), (os10, os11))

    def gather(s, p, b):
        pltpu.async_copy(table_hbm.at[idx_v.at[s // 2, pl.ds(_H0[b], _CH[b])]],
                         rows[p][b], gs[p][b])

    def wb_dst(r, b):
        return out_hbm.at[b0 + r, pl.ds(_H0[b], _CH[b]), pl.ds(0, _EMBED_DIM)]

    def quad_step(s4, carry):
        for k in range(4):
            b = k & 1
            p = (k >> 1) & 1
            s = s4 * 4 + k
            pltpu.make_async_copy(table_hbm.at[idx_v.at[0, pl.ds(0, _CH[b])]],
                                  rows[p][b], gs[p][b]).wait()
            pltpu.async_copy(rows[p][b], wb_dst(s // 2, b), os[p][b])

            @pl.when(s + 4 < _NSTEPS)
            def _():
                pltpu.make_async_copy(rows[p][b], wb_dst(0, b),
                                      os[p][b]).wait()
                gather(s + 4, p, b)
        return carry

    for k in range(4):
        gather(k, (k >> 1) & 1, k & 1)
    lax.fori_loop(0, _NSTEPS // 4, quad_step, 0)

    for p in range(2):
        for b in range(2):
            pltpu.make_async_copy(rows[p][b], wb_dst(0, b), os[p][b]).wait()


@jax.jit
def kernel(x, table):
    x2 = x * 2
    trans = functools.partial(
        pl.kernel,
        out_type=jax.ShapeDtypeStruct((_NUM_VOCAB, 2 * _EMBED_DIM),
                                      jnp.float32),
        mesh=plsc.VectorSubcoreMesh(core_axis_name="c", subcore_axis_name="s"),
        scratch_types=[
            pltpu.VMEM((_EMBED_DIM, _TBLK), jnp.float32),
            pltpu.VMEM((_EMBED_DIM, _TBLK), jnp.float32),
            pltpu.VMEM((_TBLK, 2 * _EMBED_DIM), jnp.float32),
            pltpu.VMEM((_TBLK, 2 * _EMBED_DIM), jnp.float32),
            pltpu.VMEM((_EMBED_DIM, _TAIL), jnp.float32),
        ] + [pltpu.SemaphoreType.DMA] * 4,
        compiler_params=pltpu.CompilerParams(use_tc_tiling_on_sc=True,
                                             needs_layout_passes=False),
    )(_trans_body)
    t128 = trans(table.T, table.T[:, _NFULL * _TBLK:])
    t2 = t128.reshape(2 * _NUM_VOCAB, _EMBED_DIM)
    run = functools.partial(
        pl.kernel,
        out_type=jax.ShapeDtypeStruct((_BATCH, _HIST, 2 * _EMBED_DIM),
                                      jnp.float32),
        mesh=plsc.VectorSubcoreMesh(core_axis_name="c", subcore_axis_name="s"),
        scratch_types=[
            pltpu.VMEM((_BPW, _HIST), jnp.int32),
            pltpu.VMEM((_CH[0], _EMBED_DIM), jnp.float32),
            pltpu.VMEM((_CH[1], _EMBED_DIM), jnp.float32),
            pltpu.VMEM((_CH[0], _EMBED_DIM), jnp.float32),
            pltpu.VMEM((_CH[1], _EMBED_DIM), jnp.float32),
        ] + [pltpu.SemaphoreType.DMA] * 8,
        compiler_params=pltpu.CompilerParams(use_tc_tiling_on_sc=False),
    )(_body)
    out_pad = run(x2, t2)
    return out_pad[:, :, :_EMBED_DIM]


# speedup vs baseline: 2.9364x; 1.3484x over previous
import functools

import jax
import jax.numpy as jnp
from jax import lax
from jax.experimental import pallas as pl
from jax.experimental.pallas import tpu as pltpu
from jax.experimental.pallas import tpu_sc as plsc

_NUM_VOCAB = 1000000
_EMBED_DIM = 64
_BATCH = 4096
_HIST = 200

_INFO = plsc.get_sparse_core_info()
_NC, _NS = _INFO.num_cores, _INFO.num_subcores
_NW = _NC * _NS
_BPW = _BATCH // _NW
_CH = (128, 72)
_H0 = (0, 128)


_NSTEPS = 2 * _BPW

_TBLK = 128
_NFULL = _NUM_VOCAB // _TBLK
_TAIL = _NUM_VOCAB - _NFULL * _TBLK


def _iota16():
    return lax.iota(jnp.int32, 16)


def _trans_body(tT_hbm, tail_hbm, out_hbm, inb0, inb1, outb0, outb1, tinb,
                tin0, tin1, tout0, tout1):
    wid = lax.axis_index("s") * _NC + lax.axis_index("c")
    inb = (inb0, inb1)
    outb = (outb0, outb1)
    tin = (tin0, tin1)
    tout = (tout0, tout1)
    nblk = 244 + jnp.where(wid < jnp.int32(_NFULL - 244 * _NW), 1, 0)

    def col0_of(i):
        return _TBLK * (wid + _NW * i)

    def load(i, p):
        pltpu.async_copy(tT_hbm.at[:, pl.ds(col0_of(i), _TBLK)], inb[p],
                         tin[p])

    def transpose_block(src, dst, width):
        iot = _iota16()

        def col8(j8, carry):
            vecs = []
            for u in range(8):
                j = j8 * 8 + u
                cols = lax.bitwise_and(j + iot, jnp.int32(width - 1))
                for k in range(_EMBED_DIM // 16):
                    rows = iot + 16 * k
                    vecs.append((cols, rows,
                                 plsc.load_gather(src, [rows, cols])))
            for cols, rows, vec in vecs:
                plsc.store_scatter(dst, [cols, rows], vec)
            return carry
        lax.fori_loop(0, width // 8, col8, 0)

    load(0, 0)

    @pl.when(nblk > 1)
    def _():
        load(1, 1)

    def blk_step(i, carry):
        for p in range(2):
            ii = i * 2 + p

            @pl.when(ii < nblk)
            def _():
                pltpu.make_async_copy(tT_hbm.at[:, pl.ds(0, _TBLK)], inb[p],
                                      tin[p]).wait()
                @pl.when(ii >= 2)
                def _():
                    pltpu.make_async_copy(outb[p],
                                          out_hbm.at[pl.ds(0, _TBLK)],
                                          tout[p]).wait()
                transpose_block(inb[p], outb[p], _TBLK)

                @pl.when(ii + 2 < nblk)
                def _():
                    load(ii + 2, p)
                pltpu.async_copy(outb[p], out_hbm.at[pl.ds(col0_of(ii), _TBLK)],
                                 tout[p])
        return carry

    lax.fori_loop(0, 123, blk_step, 0)

    @pl.when(nblk > 0)
    def _():
        pltpu.make_async_copy(outb[0], out_hbm.at[pl.ds(0, _TBLK)],
                              tout[0]).wait()

    @pl.when(nblk > 1)
    def _():
        pltpu.make_async_copy(outb[1], out_hbm.at[pl.ds(0, _TBLK)],
                              tout[1]).wait()

    @pl.when(wid == 0)
    def _():
        pltpu.sync_copy(tail_hbm, tinb)
        transpose_block(tinb, outb0, _TAIL)
        pltpu.sync_copy(outb0.at[pl.ds(0, _TAIL)],
                        out_hbm.at[pl.ds(_NFULL * _TBLK, _TAIL)])


def _body(x_hbm, table_hbm, out_hbm, idx_v,
          rows00, rows01, rows10, rows11,
          gs00, gs01, gs10, gs11, os00, os01, os10, os11):
    wid = lax.axis_index("s") * _NC + lax.axis_index("c")
    b0 = wid * _BPW
    pltpu.sync_copy(x_hbm.at[pl.ds(b0, _BPW)], idx_v)

    rows = ((rows00, rows01), (rows10, rows11))
    gs = ((gs00, gs01), (gs10, gs11))
    os = ((os00, os01), (os10, os11))

    def gather(s, p, b):
        pltpu.async_copy(table_hbm.at[idx_v.at[s // 2, pl.ds(_H0[b], _CH[b])]],
                         rows[p][b], gs[p][b])

    def wb_dst(r, b):
        return out_hbm.at[b0 + r, pl.ds(_H0[b], _CH[b]), pl.ds(0, _EMBED_DIM)]

    def quad_step(s4, carry):
        for k in range(4):
            b = k & 1
            p = (k >> 1) & 1
            s = s4 * 4 + k
            pltpu.make_async_copy(table_hbm.at[idx_v.at[0, pl.ds(0, _CH[b])]],
                                  rows[p][b], gs[p][b]).wait()
            pltpu.async_copy(rows[p][b], wb_dst(s // 2, b), os[p][b])

            @pl.when(s + 4 < _NSTEPS)
            def _():
                pltpu.make_async_copy(rows[p][b], wb_dst(0, b),
                                      os[p][b]).wait()
                gather(s + 4, p, b)
        return carry

    for k in range(4):
        gather(k, (k >> 1) & 1, k & 1)
    lax.fori_loop(0, _NSTEPS // 4, quad_step, 0)

    for p in range(2):
        for b in range(2):
            pltpu.make_async_copy(rows[p][b], wb_dst(0, b), os[p][b]).wait()


@jax.jit
def kernel(x, table):
    x2 = x * 2
    trans = functools.partial(
        pl.kernel,
        out_type=jax.ShapeDtypeStruct((_NUM_VOCAB, 2 * _EMBED_DIM),
                                      jnp.float32),
        mesh=plsc.VectorSubcoreMesh(core_axis_name="c", subcore_axis_name="s"),
        scratch_types=[
            pltpu.VMEM((_EMBED_DIM, _TBLK), jnp.float32),
            pltpu.VMEM((_EMBED_DIM, _TBLK), jnp.float32),
            pltpu.VMEM((_TBLK, 2 * _EMBED_DIM), jnp.float32),
            pltpu.VMEM((_TBLK, 2 * _EMBED_DIM), jnp.float32),
            pltpu.VMEM((_EMBED_DIM, _TAIL), jnp.float32),
        ] + [pltpu.SemaphoreType.DMA] * 4,
        compiler_params=pltpu.CompilerParams(use_tc_tiling_on_sc=True,
                                             needs_layout_passes=False),
    )(_trans_body)
    t128 = trans(table.T, table.T[:, _NFULL * _TBLK:])
    t2 = t128.reshape(2 * _NUM_VOCAB, _EMBED_DIM)
    run = functools.partial(
        pl.kernel,
        out_type=jax.ShapeDtypeStruct((_BATCH, _HIST, 2 * _EMBED_DIM),
                                      jnp.float32),
        mesh=plsc.VectorSubcoreMesh(core_axis_name="c", subcore_axis_name="s"),
        scratch_types=[
            pltpu.VMEM((_BPW, _HIST), jnp.int32),
            pltpu.VMEM((_CH[0], _EMBED_DIM), jnp.float32),
            pltpu.VMEM((_CH[1], _EMBED_DIM), jnp.float32),
            pltpu.VMEM((_CH[0], _EMBED_DIM), jnp.float32),
            pltpu.VMEM((_CH[1], _EMBED_DIM), jnp.float32),
        ] + [pltpu.SemaphoreType.DMA] * 8,
        compiler_params=pltpu.CompilerParams(use_tc_tiling_on_sc=False),
    )(_body)
    out_pad = run(x2, t2)
    return out_pad[:, :, :_EMBED_DIM]
